# baseline (speedup 1.0000x reference)
import jax
import jax.numpy as jnp
from jax import lax
from jax.experimental import pallas as pl
from jax.experimental.pallas import tpu as pltpu
from jax.experimental.pallas import tpu_sc as plsc

D_MODEL = 128
CHUNK = 128
NBUF = 4
NUM_WORKERS = 32


def kernel(x, table):
    B, L = x.shape
    N = B * L
    rows_per_w = N // NUM_WORKERS
    nchunks = rows_per_w // CHUNK
    idx2d = x.reshape(N // CHUNK, CHUNK)
    mesh = plsc.VectorSubcoreMesh(core_axis_name="c", subcore_axis_name="s")

    @jax.jit
    def run(table, idx2d):
        @pl.kernel(
            out_type=jax.ShapeDtypeStruct((N, D_MODEL), table.dtype),
            mesh=mesh,
            scratch_types=[
                pltpu.VMEM((nchunks, CHUNK), jnp.int32),
            ]
            + [pltpu.VMEM((CHUNK, D_MODEL), jnp.float32)] * NBUF
            + [pltpu.VMEM_SHARED((16, 2, CHUNK, D_MODEL), jnp.float32)]
            + [pltpu.SemaphoreType.DMA] * (NBUF + 4),
        )
        def gather_kernel(table_hbm, idx_hbm, out_hbm, idx_v, *scratch):
            rows = scratch[:NBUF]
            spmem = scratch[NBUF]
            gsems = scratch[NBUF + 1:2 * NBUF + 1]
            wsems = scratch[2 * NBUF + 1:2 * NBUF + 3]
            csems = scratch[2 * NBUF + 3:]
            sid = lax.axis_index("s")
            wid = lax.axis_index("s") * 2 + lax.axis_index("c")
            rbase = wid * rows_per_w
            ibase = wid * nchunks

            pltpu.sync_copy(idx_hbm.at[pl.ds(ibase, nchunks)], idx_v)

            def gather(c, b, start):
                cp = pltpu.make_async_copy(
                    table_hbm.at[idx_v.at[c]], rows[b], gsems[b]
                )
                cp.start() if start else cp.wait()

            def write(c, s, start):
                cp = pltpu.make_async_copy(
                    spmem.at[sid, s], out_hbm.at[pl.ds(rbase + c * CHUNK, CHUNK)],
                    wsems[s],
                )
                cp.start() if start else cp.wait()

            def crossbar(b, s, start):
                cp = pltpu.make_async_copy(rows[b], spmem.at[sid, s], csems[s])
                cp.start() if start else cp.wait()

            for b in range(NBUF):
                gather(b, b, True)

            @pl.loop(0, nchunks, step=NBUF)
            def _(i):
                for b in range(NBUF):
                    c = i + b
                    s = b % 2
                    gather(c, b, False)

                    @pl.when(c >= 2)
                    def _():
                        write(c - 2, s, False)

                    crossbar(b, s, True)

                    @pl.when(c >= 1)
                    def _():
                        crossbar((b - 1) % NBUF, 1 - s, False)
                        write(c - 1, 1 - s, True)

                    @pl.when((c >= 1) & (c + 3 < nchunks))
                    def _():
                        gather(c + 3, (b - 1) % NBUF, True)

            crossbar((nchunks - 1) % NBUF, (nchunks - 1) % 2, False)
            write(nchunks - 1, (nchunks - 1) % 2, True)
            write(nchunks - 2, (nchunks - 2) % 2, False)
            write(nchunks - 1, (nchunks - 1) % 2, False)

        return gather_kernel(table, idx2d)

    out = run(table, idx2d)
    return out.reshape(B, L, D_MODEL)


# speedup vs baseline: 1.0097x; 1.0097x over previous
import jax
import jax.numpy as jnp
from jax import lax
from jax.experimental import pallas as pl
from jax.experimental.pallas import tpu as pltpu
from jax.experimental.pallas import tpu_sc as plsc

D_MODEL = 128
CHUNK = 128
NBUF = 4
NUM_WORKERS = 32


def kernel(x, table):
    B, L = x.shape
    N = B * L
    rows_per_w = N // NUM_WORKERS
    nchunks = rows_per_w // CHUNK
    idx2d = x.reshape(N // CHUNK, CHUNK)
    mesh = plsc.VectorSubcoreMesh(core_axis_name="c", subcore_axis_name="s")

    @jax.jit
    def run(table, idx2d):
        @pl.kernel(
            out_type=jax.ShapeDtypeStruct((N, D_MODEL), table.dtype),
            mesh=mesh,
            scratch_types=[
                pltpu.VMEM((nchunks, CHUNK), jnp.int32),
            ]
            + [pltpu.VMEM((CHUNK, D_MODEL), jnp.float32)] * NBUF
            + [pltpu.VMEM_SHARED((16, 2, CHUNK, D_MODEL), jnp.float32)]
            + [pltpu.SemaphoreType.DMA] * (NBUF + 2),
        )
        def gather_kernel(table_hbm, idx_hbm, out_hbm, idx_v, *scratch):
            rows = scratch[:NBUF]
            spmem = scratch[NBUF]
            gsems = scratch[NBUF + 1:2 * NBUF + 1]
            wsems = scratch[2 * NBUF + 1:]
            sid = lax.axis_index("s")
            wid = lax.axis_index("s") * 2 + lax.axis_index("c")
            rbase = wid * rows_per_w
            ibase = wid * nchunks

            pltpu.sync_copy(idx_hbm.at[pl.ds(ibase, nchunks)], idx_v)

            def gather(c, b, start):
                cp = pltpu.make_async_copy(
                    table_hbm.at[idx_v.at[c]], rows[b], gsems[b]
                )
                cp.start() if start else cp.wait()

            def write(c, s, start):
                cp = pltpu.make_async_copy(
                    spmem.at[sid, s], out_hbm.at[pl.ds(rbase + c * CHUNK, CHUNK)],
                    wsems[s],
                )
                cp.start() if start else cp.wait()

            for b in range(NBUF):
                gather(b, b, True)

            @pl.loop(0, nchunks, step=NBUF)
            def _(i):
                for b in range(NBUF):
                    c = i + b
                    s = b % 2
                    gather(c, b, False)

                    @pl.when(c >= 2)
                    def _():
                        write(c - 2, s, False)

                    pltpu.sync_copy(rows[b], spmem.at[sid, s])
                    write(c, s, True)

                    @pl.when(c + NBUF < nchunks)
                    def _():
                        gather(c + NBUF, b, True)

            write(nchunks - 2, 0, False)
            write(nchunks - 1, 1, False)

        return gather_kernel(table, idx2d)

    out = run(table, idx2d)
    return out.reshape(B, L, D_MODEL)
